# baseline (speedup 1.0000x reference)
import jax
import jax.numpy as jnp
from jax import lax
from jax.experimental import pallas as pl
from jax.experimental.pallas import tpu as pltpu

EMBED_DIM = 16
DELTA_V = 0.5
DELTA_D = 3.0

_S = 4


def _loss_kernel(e_hbm, m_hbm, var_ref, dist_ref, e_sv, m_sv, en_sv, eb_sv,
                 sems, *, L, B, D, N):
    ns = N // _S
    copies = []
    k = 0
    for bi in range(B):
        per_b = []
        for s in range(_S):
            ce = pltpu.make_async_copy(
                e_hbm.at[bi, :, pl.ds(s * ns, ns)],
                e_sv.at[bi, :, pl.ds(s * ns, ns)], sems.at[k])
            cm = pltpu.make_async_copy(
                m_hbm.at[bi, :, pl.ds(s * ns, ns)],
                m_sv.at[bi, :, pl.ds(s * ns, ns)], sems.at[k + 1])
            per_b.append((ce, cm))
            k += 2
        copies.append(per_b)
    for per_b in copies:
        for ce, cm in per_b:
            ce.start()
            cm.start()

    var_total = jnp.zeros((), jnp.float32)
    dist_total = jnp.zeros((), jnp.float32)
    ones_d = jnp.ones((1, D), jnp.float32)
    for bi in range(B):
        counts = jnp.zeros((L, 1), jnp.float32)
        sums = jnp.zeros((L, D), jnp.float32)
        for s in range(_S):
            ce, cm = copies[bi][s]
            ce.wait()
            cm.wait()
            e = e_sv[bi, :, pl.ds(s * ns, ns)]
            mf = m_sv[bi, :, pl.ds(s * ns, ns)].astype(jnp.float32)
            counts += jnp.sum(mf, axis=1, keepdims=True)
            sums += lax.dot_general(
                mf, e, (((1,), (1,)), ((), ())),
                preferred_element_type=jnp.float32)
            en_sv[bi, :, pl.ds(s * ns, ns)] = lax.dot_general(
                ones_d, e * e, (((1,), (0,)), ((), ())),
                preferred_element_type=jnp.float32)
            eb_sv[bi, :, pl.ds(s * ns, ns)] = e.astype(jnp.bfloat16)
        means = sums / counts
        mnorm2 = jnp.sum(means * means, axis=1, keepdims=True)

        var_num = jnp.zeros((L, 1), jnp.float32)
        for s in range(_S):
            eb = eb_sv[bi, :, pl.ds(s * ns, ns)]
            mi = m_sv[bi, :, pl.ds(s * ns, ns)]
            enorm2 = en_sv[bi, :, pl.ds(s * ns, ns)]
            dot2 = lax.dot_general(
                means.astype(jnp.bfloat16), eb, (((1,), (0,)), ((), ())),
                preferred_element_type=jnp.float32)
            sq = jnp.maximum((enorm2 + mnorm2) - 2.0 * dot2, 0.0)
            norm = sq * lax.rsqrt(sq + 1e-20)
            var_t = jnp.where(mi > 0,
                              jnp.maximum(norm - DELTA_V, 0.0) ** 2, 0.0)
            var_num += jnp.sum(var_t, axis=1, keepdims=True)
        var_total += jnp.sum(var_num / counts) / (L * B)

        cdiff = means[:, None, :] - means[None, :, :]
        dsq = jnp.sum(cdiff * cdiff, axis=2)
        eye = (lax.broadcasted_iota(jnp.int32, (L, L), 0)
               == lax.broadcasted_iota(jnp.int32, (L, L), 1)
               ).astype(jnp.float32)
        dist = jnp.sqrt(jnp.maximum(dsq, 0.0)) + eye * DELTA_D
        dist_terms = jnp.maximum(DELTA_D - dist, 0.0) ** 2
        dist_total += jnp.sum(dist_terms) / (L * (L - 1)) / 2.0 / B

    var_ref[:, :] = var_total.reshape(1, 1)
    dist_ref[:, :] = dist_total.reshape(1, 1)


def kernel(embedding, seg_gt):
    B, D, H, W = embedding.shape
    L = seg_gt.shape[1]
    N = H * W

    e = embedding.reshape(B, D, N)
    m = seg_gt.reshape(B, L, N)

    var_loss, dist_loss = pl.pallas_call(
        lambda e_ref, m_ref, v_ref, d_ref, *scratch: _loss_kernel(
            e_ref, m_ref, v_ref, d_ref, *scratch, L=L, B=B, D=D, N=N),
        in_specs=[
            pl.BlockSpec(memory_space=pltpu.MemorySpace.HBM),
            pl.BlockSpec(memory_space=pltpu.MemorySpace.HBM),
        ],
        out_specs=[
            pl.BlockSpec(memory_space=pltpu.MemorySpace.VMEM),
            pl.BlockSpec(memory_space=pltpu.MemorySpace.VMEM),
        ],
        out_shape=[
            jax.ShapeDtypeStruct((1, 1), jnp.float32),
            jax.ShapeDtypeStruct((1, 1), jnp.float32),
        ],
        scratch_shapes=[
            pltpu.VMEM((B, D, N), jnp.float32),
            pltpu.VMEM((B, L, N), m.dtype),
            pltpu.VMEM((B, 1, N), jnp.float32),
            pltpu.VMEM((B, D, N), jnp.bfloat16),
            pltpu.SemaphoreType.DMA((2 * _S * B,)),
        ],
    )(e, m)

    reg_loss = jnp.zeros((), dtype=embedding.dtype)
    return (var_loss[0, 0], dist_loss[0, 0], reg_loss)


# speedup vs baseline: 1.0114x; 1.0114x over previous
import jax
import jax.numpy as jnp
from jax import lax
from jax.experimental import pallas as pl
from jax.experimental.pallas import tpu as pltpu

EMBED_DIM = 16
DELTA_V = 0.5
DELTA_D = 3.0

_S = 4


def _loss_kernel(e_hbm, m_hbm, var_ref, dist_ref, e_sv, m_sv, en_sv, sems,
                 *, L, B, D, N):
    ns = N // _S
    copies = []
    k = 0
    for bi in range(B):
        per_b = []
        for s in range(_S):
            ce = pltpu.make_async_copy(
                e_hbm.at[bi, :, pl.ds(s * ns, ns)],
                e_sv.at[bi, :, pl.ds(s * ns, ns)], sems.at[k])
            cm = pltpu.make_async_copy(
                m_hbm.at[bi, :, pl.ds(s * ns, ns)],
                m_sv.at[bi, :, pl.ds(s * ns, ns)], sems.at[k + 1])
            per_b.append((ce, cm))
            k += 2
        copies.append(per_b)
    for per_b in copies:
        for ce, cm in per_b:
            ce.start()
            cm.start()

    var_total = jnp.zeros((), jnp.float32)
    dist_total = jnp.zeros((), jnp.float32)
    ones_d = jnp.ones((1, D), jnp.float32)
    for bi in range(B):
        counts = jnp.zeros((L, 1), jnp.float32)
        sums = jnp.zeros((L, D), jnp.float32)
        for s in range(_S):
            ce, cm = copies[bi][s]
            ce.wait()
            cm.wait()
            e = e_sv[bi, :, pl.ds(s * ns, ns)]
            mf = m_sv[bi, :, pl.ds(s * ns, ns)].astype(jnp.float32)
            counts += jnp.sum(mf, axis=1, keepdims=True)
            sums += lax.dot_general(
                mf, e, (((1,), (1,)), ((), ())),
                preferred_element_type=jnp.float32)
            en_sv[bi, :, pl.ds(s * ns, ns)] = lax.dot_general(
                ones_d, e * e, (((1,), (0,)), ((), ())),
                preferred_element_type=jnp.float32)
        means = sums / counts
        mnorm2 = jnp.sum(means * means, axis=1, keepdims=True)

        var_num = jnp.zeros((L, 1), jnp.float32)
        for s in range(_S):
            e = e_sv[bi, :, pl.ds(s * ns, ns)]
            mi = m_sv[bi, :, pl.ds(s * ns, ns)]
            enorm2 = en_sv[bi, :, pl.ds(s * ns, ns)]
            dot2 = lax.dot_general(
                means, e, (((1,), (0,)), ((), ())),
                preferred_element_type=jnp.float32)
            sq = jnp.maximum((enorm2 + mnorm2) - 2.0 * dot2, 0.0)
            norm = sq * lax.rsqrt(sq + 1e-20)
            var_t = jnp.where(mi > 0,
                              jnp.maximum(norm - DELTA_V, 0.0) ** 2, 0.0)
            var_num += jnp.sum(var_t, axis=1, keepdims=True)
        var_total += jnp.sum(var_num / counts) / (L * B)

        cdiff = means[:, None, :] - means[None, :, :]
        dsq = jnp.sum(cdiff * cdiff, axis=2)
        eye = (lax.broadcasted_iota(jnp.int32, (L, L), 0)
               == lax.broadcasted_iota(jnp.int32, (L, L), 1)
               ).astype(jnp.float32)
        dist = jnp.sqrt(jnp.maximum(dsq, 0.0)) + eye * DELTA_D
        dist_terms = jnp.maximum(DELTA_D - dist, 0.0) ** 2
        dist_total += jnp.sum(dist_terms) / (L * (L - 1)) / 2.0 / B

    var_ref[:, :] = var_total.reshape(1, 1)
    dist_ref[:, :] = dist_total.reshape(1, 1)


def kernel(embedding, seg_gt):
    B, D, H, W = embedding.shape
    L = seg_gt.shape[1]
    N = H * W

    e = embedding.reshape(B, D, N)
    m = seg_gt.reshape(B, L, N)

    var_loss, dist_loss = pl.pallas_call(
        lambda e_ref, m_ref, v_ref, d_ref, *scratch: _loss_kernel(
            e_ref, m_ref, v_ref, d_ref, *scratch, L=L, B=B, D=D, N=N),
        in_specs=[
            pl.BlockSpec(memory_space=pltpu.MemorySpace.HBM),
            pl.BlockSpec(memory_space=pltpu.MemorySpace.HBM),
        ],
        out_specs=[
            pl.BlockSpec(memory_space=pltpu.MemorySpace.VMEM),
            pl.BlockSpec(memory_space=pltpu.MemorySpace.VMEM),
        ],
        out_shape=[
            jax.ShapeDtypeStruct((1, 1), jnp.float32),
            jax.ShapeDtypeStruct((1, 1), jnp.float32),
        ],
        scratch_shapes=[
            pltpu.VMEM((B, D, N), jnp.float32),
            pltpu.VMEM((B, L, N), m.dtype),
            pltpu.VMEM((B, 1, N), jnp.float32),
            pltpu.SemaphoreType.DMA((2 * _S * B,)),
        ],
    )(e, m)

    reg_loss = jnp.zeros((), dtype=embedding.dtype)
    return (var_loss[0, 0], dist_loss[0, 0], reg_loss)
